# baseline (speedup 1.0000x reference)
import jax
import jax.numpy as jnp
from jax import lax
from jax.experimental import pallas as pl
from jax.experimental.pallas import tpu as pltpu

_B, _C, _H, _W = 8, 64, 32, 32
_K = 8192
_N = _B * _H * _W
_HW = _H * _W
_INV_T = 10.0

_BN = 128
_NPB = _HW // _BN
_G = _N // _BN
_K2 = _K // 2


def _vq_body(xf_ref, code_ref, dist_ref, assign_ref, q_ref,
             cnta_ref, an0_ref, an1_ref, sem_ref):
    i = pl.program_id(0)

    @pl.when(i == 0)
    def _prologue():
        cbt = code_ref[...]
        s = jnp.sum(cbt * cbt, axis=0, keepdims=True)
        cnt = cbt / jnp.maximum(jnp.sqrt(s), 1e-12)
        csqt = jnp.sum(cnt * cnt, axis=0, keepdims=True)
        cnta_ref[...] = jnp.concatenate(
            [cnt, jnp.ones((1, _K), jnp.float32), csqt], axis=0)

    ft = xf_ref[0]
    s = jnp.sum(ft * ft, axis=0, keepdims=True)
    r = 1.0 / jnp.maximum(jnp.sqrt(s), 1e-12)
    fnt = ft * r
    fnt2 = fnt + fnt
    fsqt = jnp.sum(fnt * fnt, axis=0, keepdims=True)
    ones_n = jnp.ones((1, _BN), jnp.float32)

    m = lax.dot_general(cnta_ref[0:64, :], fnt2, (((0,), (0,)), ((), ())),
                        preferred_element_type=jnp.float32)

    fa_t = jnp.concatenate([-fnt2, fsqt, ones_n], axis=0)
    dist_ref[...] = lax.dot_general(jnp.transpose(fa_t), cnta_ref[...],
                                    (((1,), (0,)), ((), ())),
                                    preferred_element_type=jnp.float32)

    e2 = jnp.exp(_INV_T * m)
    s2 = jnp.sum(e2, axis=0, keepdims=True)
    an = e2 / s2

    def _copies(an_buf, sem_slot, step):
        b = step // _NPB
        off = (step % _NPB) * _BN
        return [
            pltpu.make_async_copy(
                an_buf.at[pl.ds(h * _K2, _K2), :],
                assign_ref.at[b, pl.ds(h * _K2, _K2), pl.ds(off, _BN)],
                sem_ref.at[sem_slot, h])
            for h in range(2)
        ]

    @pl.when(jnp.logical_and(i >= 2, i % 2 == 0))
    def _w0():
        for c in _copies(an0_ref, 0, i - 2):
            c.wait()

    @pl.when(jnp.logical_and(i >= 2, i % 2 == 1))
    def _w1():
        for c in _copies(an1_ref, 1, i - 2):
            c.wait()

    @pl.when(i % 2 == 0)
    def _s0():
        an0_ref[...] = an
        for c in _copies(an0_ref, 0, i):
            c.start()

    @pl.when(i % 2 == 1)
    def _s1():
        an1_ref[...] = an
        for c in _copies(an1_ref, 1, i):
            c.start()

    @pl.when(i == _G - 1)
    def _drain():
        for c in _copies(an0_ref, 0, _G - 2):
            c.wait()
        for c in _copies(an1_ref, 1, _G - 1):
            c.wait()

    e1 = jnp.exp(m)
    qs = lax.dot_general(cnta_ref[0:65, :], e1, (((1,), (0,)), ((), ())),
                         preferred_element_type=jnp.float32)
    q_ref[...] = (qs[0:64, :] / qs[64:65, :])[None]


def kernel(x, codebook, cur_iter):
    del cur_iter
    xf = x.reshape(_B, _C, _HW)
    cbt = jnp.transpose(codebook)

    dist, assign_t, q_t = pl.pallas_call(
        _vq_body,
        grid=(_G,),
        in_specs=[
            pl.BlockSpec((1, _C, _BN), lambda i: (i // _NPB, 0, i % _NPB)),
            pl.BlockSpec((_C, _K), lambda i: (0, 0)),
        ],
        out_specs=[
            pl.BlockSpec((_BN, _K), lambda i: (i, 0)),
            pl.BlockSpec(memory_space=pl.ANY),
            pl.BlockSpec((1, _C, _BN), lambda i: (i // _NPB, 0, i % _NPB)),
        ],
        out_shape=[
            jax.ShapeDtypeStruct((_N, _K), jnp.float32),
            jax.ShapeDtypeStruct((_B, _K, _HW), jnp.float32),
            jax.ShapeDtypeStruct((_B, _C, _HW), jnp.float32),
        ],
        scratch_shapes=[
            pltpu.VMEM((66, _K), jnp.float32),
            pltpu.VMEM((_K, _BN), jnp.float32),
            pltpu.VMEM((_K, _BN), jnp.float32),
            pltpu.SemaphoreType.DMA((2, 2)),
        ],
    )(xf, cbt)

    q_feat = q_t.reshape(_B, _C, _H, _W)
    assignment = assign_t.reshape(_B, _K, _H, _W)
    return q_feat, assignment, dist
